# baseline (speedup 1.0000x reference)
import functools

import jax
import jax.numpy as jnp
from jax import lax
from jax.experimental import pallas as pl
from jax.experimental.pallas import tpu as pltpu
from jax.experimental.pallas import tpu_sc as plsc

_TOP_K = 8
_NUM_EXPERTS = 64
_ROW_BLOCK = 512
_LANES = 16
_PAD_E = _NUM_EXPERTS + 1
_PAD_K = _TOP_K


def _logits_block(hs_ref, w_ref, b_ref, out_ref):
    out_ref[...] = (
        jnp.dot(hs_ref[...], w_ref[...], preferred_element_type=jnp.float32)
        + b_ref[...]
    )


def _tc_logits(hs, w, bias2d):
    n_rows, hidden_dim = hs.shape
    grid = (n_rows // _ROW_BLOCK,)
    return pl.pallas_call(
        _logits_block,
        grid=grid,
        in_specs=[
            pl.BlockSpec((_ROW_BLOCK, hidden_dim), lambda i: (i, 0)),
            pl.BlockSpec((hidden_dim, _NUM_EXPERTS), lambda i: (0, 0)),
            pl.BlockSpec((1, _NUM_EXPERTS), lambda i: (0, 0)),
        ],
        out_specs=pl.BlockSpec((_ROW_BLOCK, _NUM_EXPERTS), lambda i: (i, 0)),
        out_shape=jax.ShapeDtypeStruct((n_rows, _NUM_EXPERTS), jnp.float32),
        compiler_params=pltpu.CompilerParams(
            dimension_semantics=("arbitrary",),
        ),
    )(hs, w, bias2d)


def _splat_i32(x):
    return jnp.full((_LANES,), x, dtype=jnp.int32)


def _sc_router(logits_flat, n_rows):
    nc, ns = 2, 16
    nw = nc * ns
    rows_per_w = n_rows // nw
    groups_per_w = rows_per_w // _LANES
    scores_per_w = rows_per_w * _NUM_EXPERTS
    idx_per_w = rows_per_w * _TOP_K

    mesh = plsc.VectorSubcoreMesh(core_axis_name="c", subcore_axis_name="s")

    @functools.partial(
        pl.kernel,
        out_type=[
            jax.ShapeDtypeStruct((n_rows * _NUM_EXPERTS,), jnp.float32),
            jax.ShapeDtypeStruct((n_rows * _TOP_K,), jnp.int32),
        ],
        mesh=mesh,
        compiler_params=pltpu.CompilerParams(needs_layout_passes=False),
        scratch_types=[
            pltpu.VMEM((rows_per_w * _NUM_EXPERTS,), jnp.float32),
            pltpu.VMEM((rows_per_w * _PAD_E,), jnp.float32),
            pltpu.VMEM((rows_per_w * _NUM_EXPERTS,), jnp.float32),
            pltpu.VMEM((rows_per_w * _TOP_K,), jnp.int32),
        ],
    )
    def sc_kernel(logits_hbm, scores_hbm, idx_hbm, logits_v, logits_p, scores_v, idx_v):
        wid = lax.axis_index("s") * nc + lax.axis_index("c")
        pltpu.sync_copy(
            logits_hbm.at[pl.ds(wid * scores_per_w, scores_per_w)], logits_v
        )

        lane = lax.iota(jnp.int32, _LANES)
        zeros16 = jnp.zeros((_LANES,), dtype=jnp.float32)
        neg_inf = jnp.float32(-jnp.inf)

        @plsc.parallel_loop(0, groups_per_w, step=1, unroll=2)
        def group_body(g):
            row_ids = g * _LANES + lane
            prow_base = row_ids * _PAD_E

            for r in range(_LANES):
                row = g * _LANES + r
                for q in range(4):
                    plsc.store_scatter(
                        logits_p,
                        [row * _PAD_E + 16 * q + lane],
                        logits_v[pl.ds(row * _NUM_EXPERTS + 16 * q, 16)],
                    )

            val = [jnp.full((_LANES,), neg_inf, dtype=jnp.float32)
                   for _ in range(_TOP_K)]
            idx = [_splat_i32(0) for _ in range(_TOP_K)]
            for e in range(_NUM_EXPERTS):
                v = plsc.load_gather(logits_p, [prow_base + e])
                es = _splat_i32(e)
                ge = [v > val[j] for j in range(_TOP_K)]
                new_val = list(val)
                new_idx = list(idx)
                for j in range(_TOP_K - 1, 0, -1):
                    new_val[j] = jnp.where(
                        ge[j], jnp.where(ge[j - 1], val[j - 1], v), val[j]
                    )
                    new_idx[j] = jnp.where(
                        ge[j], jnp.where(ge[j - 1], idx[j - 1], es), idx[j]
                    )
                new_val[0] = jnp.where(ge[0], v, val[0])
                new_idx[0] = jnp.where(ge[0], es, idx[0])
                val, idx = new_val, new_idx

            m = val[0]
            ex = [jnp.exp(val[j] - m) for j in range(_TOP_K)]
            denom = ex[0]
            for j in range(1, _TOP_K):
                denom = denom + ex[j]
            inv = 1.0 / denom

            for r in range(_LANES):
                for q in range(4):
                    scores_v[
                        pl.ds((g * _LANES + r) * _NUM_EXPERTS + 16 * q, 16)
                    ] = zeros16
            row_base = row_ids * _NUM_EXPERTS
            idx_base = row_ids * _TOP_K
            for j in range(_TOP_K):
                plsc.store_scatter(scores_v, [row_base + idx[j]], ex[j] * inv)
                plsc.store_scatter(idx_v, [idx_base + j], idx[j])

        pltpu.sync_copy(
            scores_v, scores_hbm.at[pl.ds(wid * scores_per_w, scores_per_w)]
        )
        pltpu.sync_copy(idx_v, idx_hbm.at[pl.ds(wid * idx_per_w, idx_per_w)])

    return sc_kernel(logits_flat)


def kernel(hidden_states, kernel, bias):
    hidden_dim = hidden_states.shape[-1]
    hs = hidden_states.reshape(-1, hidden_dim)
    n_rows = hs.shape[0]
    bias2d = bias.reshape(1, _NUM_EXPERTS)
    logits = _tc_logits(hs, kernel, bias2d)
    scores_flat, idx_flat = _sc_router(logits.reshape(-1), n_rows)
    return (
        scores_flat.reshape(n_rows, _NUM_EXPERTS),
        idx_flat.reshape(n_rows, _TOP_K),
    )


# speedup vs baseline: 1.1042x; 1.1042x over previous
import functools

import jax
import jax.numpy as jnp
from jax import lax
from jax.experimental import pallas as pl
from jax.experimental.pallas import tpu as pltpu
from jax.experimental.pallas import tpu_sc as plsc

_TOP_K = 8
_NUM_EXPERTS = 64
_ROW_BLOCK = 512
_LANES = 16
_PAD_E = _NUM_EXPERTS + 1


def _logits_block(hs_ref, w_ref, b_ref, out_ref):
    out_ref[...] = (
        jnp.dot(hs_ref[...], w_ref[...], preferred_element_type=jnp.float32)
        + b_ref[...]
    )


def _tc_logits(hs, w, bias2d):
    n_rows, hidden_dim = hs.shape
    grid = (n_rows // _ROW_BLOCK,)
    return pl.pallas_call(
        _logits_block,
        grid=grid,
        in_specs=[
            pl.BlockSpec((_ROW_BLOCK, hidden_dim), lambda i: (i, 0)),
            pl.BlockSpec((hidden_dim, _NUM_EXPERTS), lambda i: (0, 0)),
            pl.BlockSpec((1, _NUM_EXPERTS), lambda i: (0, 0)),
        ],
        out_specs=pl.BlockSpec((_ROW_BLOCK, _NUM_EXPERTS), lambda i: (i, 0)),
        out_shape=jax.ShapeDtypeStruct((n_rows, _NUM_EXPERTS), jnp.float32),
        compiler_params=pltpu.CompilerParams(
            dimension_semantics=("arbitrary",),
        ),
    )(hs, w, bias2d)


def _splat_i32(x):
    return jnp.full((_LANES,), x, dtype=jnp.int32)


def _sc_router(logits_flat, n_rows):
    nc, ns = 2, 16
    nw = nc * ns
    rows_per_w = n_rows // nw
    groups_per_w = rows_per_w // _LANES
    scores_per_w = rows_per_w * _NUM_EXPERTS
    idx_per_w = rows_per_w * _TOP_K

    mesh = plsc.VectorSubcoreMesh(core_axis_name="c", subcore_axis_name="s")

    @functools.partial(
        pl.kernel,
        out_type=[
            jax.ShapeDtypeStruct((n_rows * _NUM_EXPERTS,), jnp.float32),
            jax.ShapeDtypeStruct((n_rows * _TOP_K,), jnp.int32),
        ],
        mesh=mesh,
        compiler_params=pltpu.CompilerParams(needs_layout_passes=False),
        scratch_types=[
            pltpu.VMEM((rows_per_w * _NUM_EXPERTS,), jnp.float32),
            pltpu.VMEM((rows_per_w * _PAD_E,), jnp.float32),
            pltpu.VMEM((rows_per_w * _NUM_EXPERTS,), jnp.float32),
            pltpu.VMEM((rows_per_w * _TOP_K,), jnp.int32),
        ],
    )
    def sc_kernel(logits_hbm, scores_hbm, idx_hbm, logits_v, logits_p, scores_v, idx_v):
        wid = lax.axis_index("s") * nc + lax.axis_index("c")
        pltpu.sync_copy(
            logits_hbm.at[pl.ds(wid * scores_per_w, scores_per_w)], logits_v
        )

        lane = lax.iota(jnp.int32, _LANES)
        zeros16 = jnp.zeros((_LANES,), dtype=jnp.float32)
        neg_inf = jnp.float32(-jnp.inf)

        @plsc.parallel_loop(0, groups_per_w, step=1)
        def group_body(g):
            row_ids = g * _LANES + lane
            prow_base = row_ids * _PAD_E

            for r in range(_LANES):
                row = g * _LANES + r
                for q in range(4):
                    plsc.store_scatter(
                        logits_p,
                        [row * _PAD_E + 16 * q + lane],
                        logits_v[pl.ds(row * _NUM_EXPERTS + 16 * q, 16)],
                    )

            val = [jnp.full((_LANES,), neg_inf, dtype=jnp.float32)
                   for _ in range(_TOP_K)]
            idx = [_splat_i32(0) for _ in range(_TOP_K)]
            for e in range(_NUM_EXPERTS):
                v = plsc.load_gather(logits_p, [prow_base + e])
                es = _splat_i32(e)
                ge = [v > val[j] for j in range(_TOP_K)]
                new_val = list(val)
                new_idx = list(idx)
                for j in range(_TOP_K - 1, 0, -1):
                    new_val[j] = jnp.where(
                        ge[j], jnp.where(ge[j - 1], val[j - 1], v), val[j]
                    )
                    new_idx[j] = jnp.where(
                        ge[j], jnp.where(ge[j - 1], idx[j - 1], es), idx[j]
                    )
                new_val[0] = jnp.where(ge[0], v, val[0])
                new_idx[0] = jnp.where(ge[0], es, idx[0])
                val, idx = new_val, new_idx

            m = val[0]
            ex = [jnp.exp(val[j] - m) for j in range(_TOP_K)]
            denom = ex[0]
            for j in range(1, _TOP_K):
                denom = denom + ex[j]
            inv = 1.0 / denom

            for r in range(_LANES):
                for q in range(4):
                    scores_v[
                        pl.ds((g * _LANES + r) * _NUM_EXPERTS + 16 * q, 16)
                    ] = zeros16
            row_base = row_ids * _NUM_EXPERTS
            idx_base = row_ids * _TOP_K
            for j in range(_TOP_K):
                plsc.store_scatter(scores_v, [row_base + idx[j]], ex[j] * inv)
                plsc.store_scatter(idx_v, [idx_base + j], idx[j])

        pltpu.sync_copy(
            scores_v, scores_hbm.at[pl.ds(wid * scores_per_w, scores_per_w)]
        )
        pltpu.sync_copy(idx_v, idx_hbm.at[pl.ds(wid * idx_per_w, idx_per_w)])

    return sc_kernel(logits_flat)


def kernel(hidden_states, kernel, bias):
    hidden_dim = hidden_states.shape[-1]
    hs = hidden_states.reshape(-1, hidden_dim)
    n_rows = hs.shape[0]
    bias2d = bias.reshape(1, _NUM_EXPERTS)
    logits = _tc_logits(hs, kernel, bias2d)
    scores_flat, idx_flat = _sc_router(logits.reshape(-1), n_rows)
    return (
        scores_flat.reshape(n_rows, _NUM_EXPERTS),
        idx_flat.reshape(n_rows, _TOP_K),
    )
